# baseline (speedup 1.0000x reference)
import jax
import jax.numpy as jnp
from jax import lax
from jax.experimental import pallas as pl
from jax.experimental.pallas import tpu as pltpu
from jax.experimental.pallas import tpu_sc as plsc

_EMB = 64
_B = 1024
_L = 200

_NC = 2
_NS = 16
_NW = _NC * _NS
_ROWS_PER_W = (_B * _L) // _NW
_CHUNK = _L // 2
_NCHUNK = _ROWS_PER_W // _CHUNK
_NSLOT = 12
_PREFETCH = 8


def _pe_table() -> jax.Array:
    seq_index = jnp.arange(_L, dtype=jnp.float32).reshape(-1, 1)
    even_index = jnp.arange(0, _EMB, 2)
    denominator = jnp.power(10000.0, even_index.astype(jnp.float32) / _EMB)
    args_sc = seq_index / denominator
    pe = jnp.zeros((_L, _EMB), dtype=jnp.float32)
    pe = pe.at[:, even_index].set(jnp.sin(args_sc))
    pe = pe.at[:, even_index + 1].set(jnp.cos(args_sc))
    return pe


def _body(table_hbm, idx_hbm, pe_hbm, out_hbm, idx_v, pe_v, rows_v, *sems):
    gsem = sems[:_NSLOT]
    osem = sems[_NSLOT:]
    wid = lax.axis_index("s") * _NC + lax.axis_index("c")

    pltpu.sync_copy(idx_hbm.at[wid], idx_v)
    pltpu.sync_copy(pe_hbm, pe_v)

    def start_gather(c):
        s = c % _NSLOT
        pltpu.make_async_copy(
            table_hbm.at[idx_v.at[c]], rows_v.at[s], gsem[s]).start()

    def wait_gather(c):
        s = c % _NSLOT
        pltpu.make_async_copy(
            table_hbm.at[idx_v.at[c]], rows_v.at[s], gsem[s]).wait()

    def out_copy(c):
        s = c % _NSLOT
        row0 = wid * _ROWS_PER_W + c * _CHUNK
        return pltpu.make_async_copy(
            rows_v.at[s], out_hbm.at[pl.ds(row0, _CHUNK)], osem[s])

    for c in range(_PREFETCH):
        start_gather(c)

    for c in range(_NCHUNK):
        s = c % _NSLOT
        wait_gather(c)

        pbase = (c % 2) * _CHUNK

        @pl.loop(0, _CHUNK)
        def _row(r, s=s, pbase=pbase):
            for k in range(_EMB // 16):
                sl = pl.ds(k * 16, 16)
                rows_v[s, r, sl] = rows_v[s, r, sl] + pe_v[pbase + r, sl]

        out_copy(c).start()

        nc = c + _PREFETCH
        if nc < _NCHUNK:
            if nc >= _NSLOT:
                out_copy(nc - _NSLOT).wait()
            start_gather(nc)

    for c in range(_NCHUNK - _NSLOT, _NCHUNK):
        out_copy(c).wait()


def kernel(X, table):
    one = (X[0, 0] * 0 + 1).astype(jnp.float32)
    idx = X.reshape(_NW, _NCHUNK, _CHUNK)
    pe = _pe_table()
    mesh = plsc.VectorSubcoreMesh(core_axis_name="c", subcore_axis_name="s")
    out = pl.kernel(
        _body,
        out_type=jax.ShapeDtypeStruct((_B * _L, _EMB), jnp.float32),
        mesh=mesh,
        scratch_types=[
            pltpu.VMEM((_NCHUNK, _CHUNK), jnp.int32),
            pltpu.VMEM((_L, _EMB), jnp.float32),
            pltpu.VMEM((_NSLOT, _CHUNK, _EMB), jnp.float32),
        ] + [pltpu.SemaphoreType.DMA] * (2 * _NSLOT),
        compiler_params=pltpu.CompilerParams(use_tc_tiling_on_sc=False),
    )(table * one, idx, pe)
    return (out * one).reshape(_B, _L, _EMB)


# speedup vs baseline: 1.2958x; 1.2958x over previous
import jax
import jax.numpy as jnp
from jax import lax
from jax.experimental import pallas as pl
from jax.experimental.pallas import tpu as pltpu
from jax.experimental.pallas import tpu_sc as plsc

_EMB = 64
_B = 1024
_L = 200
_PACK = 2 * _EMB

_NC = 2
_NS = 16
_NW = _NC * _NS
_ROWS_PER_W = (_B * _L) // _NW
_CHUNK = _L
_NCHUNK = _ROWS_PER_W // _CHUNK


def _pe_table() -> jax.Array:
    seq_index = jnp.arange(_L, dtype=jnp.float32).reshape(-1, 1)
    even_index = jnp.arange(0, _EMB, 2)
    denominator = jnp.power(10000.0, even_index.astype(jnp.float32) / _EMB)
    args_sc = seq_index / denominator
    pe = jnp.zeros((_L, _EMB), dtype=jnp.float32)
    pe = pe.at[:, even_index].set(jnp.sin(args_sc))
    pe = pe.at[:, even_index + 1].set(jnp.cos(args_sc))
    return pe.reshape(_L // 2, _PACK)


def _body(tbl_hbm, idx_hbm, pe_hbm, out_hbm, idx_v, pe_v, rows_v,
          gsem0, gsem1, osem0, osem1):
    gsem = (gsem0, gsem1)
    osem = (osem0, osem1)
    wid = lax.axis_index("s") * _NC + lax.axis_index("c")
    base = wid * _ROWS_PER_W
    seq0 = wid * _NCHUNK

    pltpu.sync_copy(idx_hbm.at[pl.ds(base, _ROWS_PER_W)],
                    idx_v.at[pl.ds(0, _ROWS_PER_W)])
    pltpu.sync_copy(pe_hbm, pe_v)

    def issue_group(c, g, n, s):
        iv = idx_v[pl.ds(c * _CHUNK + g * 16, 16)]
        for i in range(n):
            pltpu.make_async_copy(
                tbl_hbm.at[pl.ds(iv[i], 1)],
                rows_v.at[s, pl.ds(g * 16 + i, 1)], gsem[s]).start()

    def issue_rows(c, s):
        @pl.loop(0, 12)
        def _grp(g, c=c, s=s):
            issue_group(c, g, 16, s)

        issue_group(c, 12, 8, s)

    def drain_rows(s):
        pltpu.make_async_copy(
            tbl_hbm.at[pl.ds(0, _CHUNK)], rows_v.at[s], gsem[s]).wait()

    def out_copy(c, s):
        return pltpu.make_async_copy(
            rows_v.at[s], out_hbm.at[seq0 + c], osem[s])

    issue_rows(0, 0)

    @pl.loop(0, _NCHUNK // 2)
    def _pair(g2):
        for b in range(2):
            c = 2 * g2 + b
            nb = 1 - b

            @pl.when(c + 1 < _NCHUNK)
            def _(b=b, c=c, nb=nb):
                @pl.when(c >= 1)
                def _():
                    out_copy(c - 1, nb).wait()

                issue_rows(c + 1, nb)

            drain_rows(b)

            @pl.loop(0, _L // 2)
            def _pairrow(q, b=b):
                for k in range(_EMB // 16):
                    sl = pl.ds(k * 16, 16)
                    sh = pl.ds(_EMB + k * 16, 16)
                    rows_v[b, 2 * q, sl] = rows_v[b, 2 * q, sl] + pe_v[q, sl]
                    rows_v[b, 2 * q + 1, sl] = (
                        rows_v[b, 2 * q + 1, sl] + pe_v[q, sh])

            out_copy(c, b).start()

    out_copy(_NCHUNK - 2, 0).wait()
    out_copy(_NCHUNK - 1, 1).wait()


def kernel(X, table):
    idx = X.reshape(_B * _L)
    pe = _pe_table()
    mesh = plsc.VectorSubcoreMesh(core_axis_name="c", subcore_axis_name="s")
    out = pl.kernel(
        _body,
        out_type=jax.ShapeDtypeStruct((_B, _L, _EMB), jnp.float32),
        mesh=mesh,
        scratch_types=[
            pltpu.VMEM((_ROWS_PER_W + 16,), jnp.int32),
            pltpu.VMEM((_L // 2, _PACK), jnp.float32),
            pltpu.VMEM((2, _CHUNK, _EMB), jnp.float32),
            pltpu.SemaphoreType.DMA,
            pltpu.SemaphoreType.DMA,
            pltpu.SemaphoreType.DMA,
            pltpu.SemaphoreType.DMA,
        ],
        compiler_params=pltpu.CompilerParams(
            use_tc_tiling_on_sc=True,
            disable_bounds_checks=True,
            disable_semaphore_checks=True,
        ),
    )(table, idx, pe)
    return out
